# baseline (speedup 1.0000x reference)
import jax
import jax.numpy as jnp
from jax import lax
from jax.experimental import pallas as pl
from jax.experimental.pallas import tpu as pltpu
from jax.experimental.pallas import tpu_sc as plsc

_N = 10000
_E = 320000
_F = 128
_H = 128
_CLS = 16

_NC = 2
_NS = 16
_NW = _NC * _NS

_DEPTH = 4
_CHUNK = 48
_NCHUNK = 212
_E_PAD = _NW * _NCHUNK * _CHUNK

_NPAD = 10240
_STRIPE = _NPAD // _NS
_TRASH = _NPAD - 1

_vmesh = plsc.VectorSubcoreMesh(core_axis_name="c", subcore_axis_name="s")


def _deg_body(dst_hbm, zeros_hbm, out_hbm, dstv, ones_v, deg_sh):
    cid = lax.axis_index("c")
    sid = lax.axis_index("s")
    wid = cid * _NS + sid
    stripe = pl.ds(sid * _STRIPE, _STRIPE)
    pltpu.sync_copy(zeros_hbm.at[stripe], deg_sh.at[stripe])
    @pl.loop(0, _CHUNK // 16)
    def _(i):
        ones_v.at[pl.ds(i * 16, 16)][...] = jnp.ones((16,), jnp.float32)
    pltpu.sync_copy(dst_hbm.at[wid], dstv)
    plsc.subcore_barrier()
    @pl.loop(0, _NCHUNK)
    def _(c):
        pltpu.sync_copy(ones_v, deg_sh.at[dstv.at[c]], add=True)
    plsc.subcore_barrier()
    pltpu.sync_copy(deg_sh.at[stripe], out_hbm.at[cid].at[stripe])


def _make_agg(d, staged):
    n_h = _NPAD if staged else _N

    def body(h_hbm, src_hbm, dst_hbm, zeros_hbm, out_hbm, *rest):
        srcv, dstv = rest[0], rest[1]
        rows = rest[2:2 + _DEPTH]
        s_sh = rest[2 + _DEPTH]
        rest = rest[3 + _DEPTH:]
        if staged:
            h_sh = rest[0]
            rest = rest[1:]
        gsems = rest[:_DEPTH]
        ssems = rest[_DEPTH:]
        cid = lax.axis_index("c")
        sid = lax.axis_index("s")
        wid = cid * _NS + sid
        stripe = pl.ds(sid * _STRIPE, _STRIPE)
        pltpu.sync_copy(zeros_hbm.at[stripe], s_sh.at[stripe])
        if staged:
            pltpu.sync_copy(h_hbm.at[stripe], h_sh.at[stripe])
        pltpu.sync_copy(src_hbm.at[wid], srcv)
        pltpu.sync_copy(dst_hbm.at[wid], dstv)
        plsc.subcore_barrier()
        gsrc = h_sh if staged else h_hbm

        def issue_g(k, b):
            pltpu.async_copy(gsrc.at[srcv.at[k]], rows[b], gsems[b])

        def drain_g(k, b):
            pltpu.make_async_copy(gsrc.at[srcv.at[k]], rows[b],
                                  gsems[b]).wait()

        def issue_s(k, b):
            pltpu.async_copy(rows[b], s_sh.at[dstv.at[k]], ssems[b],
                             add=True)

        def drain_s(k, b):
            pltpu.make_async_copy(rows[b], s_sh.at[dstv.at[k]],
                                  ssems[b]).wait()

        for j in range(_DEPTH - 1):
            issue_g(j, j)
        @pl.loop(0, _NCHUNK, step=_DEPTH)
        def _(c):
            for j in range(_DEPTH):
                k = c + j
                b = j
                bn = (j + _DEPTH - 1) % _DEPTH
                drain_g(k, b)
                issue_s(k, b)
                @pl.when(k + _DEPTH - 1 < _NCHUNK)
                def _():
                    @pl.when(k > 0)
                    def _():
                        drain_s(k - 1, bn)
                    issue_g(k + _DEPTH - 1, bn)
        for j in range(_DEPTH):
            drain_s(_NCHUNK - _DEPTH + j, j)
        plsc.subcore_barrier()
        pltpu.sync_copy(s_sh.at[stripe], out_hbm.at[cid].at[stripe])

    return pl.kernel(
        body,
        out_type=jax.ShapeDtypeStruct((_NC, _NPAD, d), jnp.float32),
        mesh=_vmesh,
        scratch_types=[
            pltpu.VMEM((_NCHUNK, _CHUNK), jnp.int32),
            pltpu.VMEM((_NCHUNK, _CHUNK), jnp.int32),
        ] + [pltpu.VMEM((_CHUNK, d), jnp.float32)] * _DEPTH + [
            pltpu.VMEM_SHARED((_NPAD, d), jnp.float32),
        ] + ([pltpu.VMEM_SHARED((_NPAD, d), jnp.float32)] if staged else []) + [
            pltpu.SemaphoreType.DMA] * (2 * _DEPTH),
        compiler_params=pltpu.CompilerParams(use_tc_tiling_on_sc=False),
    )


_deg_call = pl.kernel(
    _deg_body,
    out_type=jax.ShapeDtypeStruct((_NC, _NPAD), jnp.float32),
    mesh=_vmesh,
    scratch_types=[
        pltpu.VMEM((_NCHUNK, _CHUNK), jnp.int32),
        pltpu.VMEM((_CHUNK,), jnp.float32),
        pltpu.VMEM_SHARED((_NPAD,), jnp.float32),
    ],
)

_agg_call_h = _make_agg(_H, staged=False)
_agg_call_c = _make_agg(_CLS, staged=True)


def _dinv_from(degT_ref):
    deg = degT_ref[:, 0:1] + degT_ref[:, 1:2] + 1.0
    return lax.rsqrt(deg)


def _tc_g_body(x_ref, w1_ref, g_ref):
    g_ref[...] = jnp.dot(x_ref[...], w1_ref[...],
                         preferred_element_type=jnp.float32)


def _tc_scale_body(g_ref, degT_ref, h_ref):
    h_ref[...] = g_ref[...] * _dinv_from(degT_ref)


def _tc_d_body(s_ref, h1_ref, degT_ref, b1_ref, w2_ref, out_ref):
    dinv = _dinv_from(degT_ref)
    s = s_ref[0, :_N] + s_ref[1, :_N] + h1_ref[...]
    z = jnp.maximum(dinv * s + b1_ref[...], 0.0)
    h2 = jnp.dot(z, w2_ref[...], preferred_element_type=jnp.float32)
    out_ref[:_N] = h2 * dinv
    out_ref[_N:] = jnp.zeros((_NPAD - _N, _CLS), jnp.float32)


def _tc_f_body(s2_ref, h2_ref, degT_ref, b2_ref, out_ref):
    dinv = _dinv_from(degT_ref)
    o = dinv * (s2_ref[0, :_N] + s2_ref[1, :_N] + h2_ref[:_N]) + b2_ref[...]
    m = jnp.max(o, axis=1, keepdims=True)
    lse = jnp.log(jnp.sum(jnp.exp(o - m), axis=1, keepdims=True)) + m
    out_ref[...] = o - lse


_tc_g = pl.pallas_call(
    _tc_g_body, out_shape=jax.ShapeDtypeStruct((_N, _H), jnp.float32))
_tc_scale = pl.pallas_call(
    _tc_scale_body, out_shape=jax.ShapeDtypeStruct((_N, _H), jnp.float32))
_tc_d = pl.pallas_call(
    _tc_d_body, out_shape=jax.ShapeDtypeStruct((_NPAD, _CLS), jnp.float32))
_tc_f = pl.pallas_call(
    _tc_f_body, out_shape=jax.ShapeDtypeStruct((_N, _CLS), jnp.float32))


@jax.jit
def _run(x, edge_index, W1, b1, W2, b2):
    ei = edge_index.astype(jnp.int32)
    pad = _E_PAD - _E
    pad_ids = jnp.arange(pad, dtype=jnp.int32)
    src = jnp.concatenate([ei[0], pad_ids % _N])
    dst = jnp.concatenate([ei[1], _N + pad_ids % (_NPAD - _N)])
    src = src.reshape(_NW, _NCHUNK, _CHUNK)
    dst = dst.reshape(_NW, _NCHUNK, _CHUNK)
    z_deg = jnp.zeros((_NPAD,), jnp.float32)
    z_h = jnp.zeros((_NPAD, _H), jnp.float32)
    z_c = jnp.zeros((_NPAD, _CLS), jnp.float32)

    g = _tc_g(x, W1)
    deg_parts = _deg_call(dst, z_deg)
    degT = deg_parts[:, :_N].T

    h1t = _tc_scale(g, degT)
    s1 = _agg_call_h(h1t, src, dst, z_h)
    h2t = _tc_d(s1, h1t, degT, b1.reshape(1, _H), W2)
    s2 = _agg_call_c(h2t, src, dst, z_c)
    return _tc_f(s2, h2t, degT, b2.reshape(1, _CLS))


def kernel(x, edge_index, W1, b1, W2, b2):
    return _run(x, edge_index, W1, b1, W2, b2)


# speedup vs baseline: 1.0593x; 1.0593x over previous
import jax
import jax.numpy as jnp
from jax import lax
from jax.experimental import pallas as pl
from jax.experimental.pallas import tpu as pltpu
from jax.experimental.pallas import tpu_sc as plsc

_N = 10000
_E = 320000
_F = 128
_H = 128
_CLS = 16

_NC = 2
_NS = 16
_NW = _NC * _NS

_DEPTH = 3
_CHUNK1 = 72
_NCHUNK1 = 141
_CHUNK2 = 128
_NCHUNK2 = 81

_NPAD = 10240
_STRIPE = _NPAD // _NS
_TRASH = _NPAD - 1

_vmesh = plsc.VectorSubcoreMesh(core_axis_name="c", subcore_axis_name="s")


def _deg_body(dst_hbm, zeros_hbm, out_hbm, dstv, ones_v, deg_sh):
    cid = lax.axis_index("c")
    sid = lax.axis_index("s")
    wid = cid * _NS + sid
    stripe = pl.ds(sid * _STRIPE, _STRIPE)
    pltpu.sync_copy(zeros_hbm.at[stripe], deg_sh.at[stripe])
    @pl.loop(0, _CHUNK2 // 16)
    def _(i):
        ones_v.at[pl.ds(i * 16, 16)][...] = jnp.ones((16,), jnp.float32)
    pltpu.sync_copy(dst_hbm.at[wid], dstv)
    plsc.subcore_barrier()
    @pl.loop(0, _NCHUNK2)
    def _(c):
        pltpu.sync_copy(ones_v, deg_sh.at[dstv.at[c]], add=True)
    plsc.subcore_barrier()
    pltpu.sync_copy(deg_sh.at[stripe], out_hbm.at[cid].at[stripe])


def _make_agg(d, staged, chunk, nchunk):
    def body(h_hbm, src_hbm, dst_hbm, zeros_hbm, out_hbm, *rest):
        srcv, dstv = rest[0], rest[1]
        rows = rest[2:2 + _DEPTH]
        s_sh = rest[2 + _DEPTH]
        rest = rest[3 + _DEPTH:]
        if staged:
            h_sh = rest[0]
            rest = rest[1:]
        gsems = rest[:_DEPTH]
        ssems = rest[_DEPTH:]
        cid = lax.axis_index("c")
        sid = lax.axis_index("s")
        wid = cid * _NS + sid
        stripe = pl.ds(sid * _STRIPE, _STRIPE)
        pltpu.sync_copy(zeros_hbm.at[stripe], s_sh.at[stripe])
        if staged:
            pltpu.sync_copy(h_hbm.at[stripe], h_sh.at[stripe])
        pltpu.sync_copy(src_hbm.at[wid], srcv)
        pltpu.sync_copy(dst_hbm.at[wid], dstv)
        plsc.subcore_barrier()
        gsrc = h_sh if staged else h_hbm

        def issue_g(k, b):
            pltpu.async_copy(gsrc.at[srcv.at[k]], rows[b], gsems[b])

        def drain_g(k, b):
            pltpu.make_async_copy(gsrc.at[srcv.at[k]], rows[b],
                                  gsems[b]).wait()

        def issue_s(k, b):
            pltpu.async_copy(rows[b], s_sh.at[dstv.at[k]], ssems[b],
                             add=True)

        def drain_s(k, b):
            pltpu.make_async_copy(rows[b], s_sh.at[dstv.at[k]],
                                  ssems[b]).wait()

        for j in range(_DEPTH - 1):
            issue_g(j, j)
        @pl.loop(0, nchunk, step=_DEPTH)
        def _(c):
            for j in range(_DEPTH):
                k = c + j
                b = j
                bn = (j + _DEPTH - 1) % _DEPTH
                drain_g(k, b)
                issue_s(k, b)
                @pl.when(k + _DEPTH - 1 < nchunk)
                def _():
                    @pl.when(k > 0)
                    def _():
                        drain_s(k - 1, bn)
                    issue_g(k + _DEPTH - 1, bn)
        for j in range(_DEPTH):
            drain_s(nchunk - _DEPTH + j, j)
        plsc.subcore_barrier()
        pltpu.sync_copy(s_sh.at[stripe], out_hbm.at[cid].at[stripe])

    return pl.kernel(
        body,
        out_type=jax.ShapeDtypeStruct((_NC, _NPAD, d), jnp.float32),
        mesh=_vmesh,
        scratch_types=[
            pltpu.VMEM((nchunk, chunk), jnp.int32),
            pltpu.VMEM((nchunk, chunk), jnp.int32),
        ] + [pltpu.VMEM((chunk, d), jnp.float32)] * _DEPTH + [
            pltpu.VMEM_SHARED((_NPAD, d), jnp.float32),
        ] + ([pltpu.VMEM_SHARED((_NPAD, d), jnp.float32)] if staged else []) + [
            pltpu.SemaphoreType.DMA] * (2 * _DEPTH),
        compiler_params=pltpu.CompilerParams(use_tc_tiling_on_sc=False),
    )


_deg_call = pl.kernel(
    _deg_body,
    out_type=jax.ShapeDtypeStruct((_NC, _NPAD), jnp.float32),
    mesh=_vmesh,
    scratch_types=[
        pltpu.VMEM((_NCHUNK2, _CHUNK2), jnp.int32),
        pltpu.VMEM((_CHUNK2,), jnp.float32),
        pltpu.VMEM_SHARED((_NPAD,), jnp.float32),
    ],
)

_agg_call_h = _make_agg(_H, staged=False, chunk=_CHUNK1, nchunk=_NCHUNK1)
_agg_call_c = _make_agg(_CLS, staged=True, chunk=_CHUNK2, nchunk=_NCHUNK2)


def _dinv_from(degT_ref):
    deg = degT_ref[:, 0:1] + degT_ref[:, 1:2] + 1.0
    return lax.rsqrt(deg)


def _tc_g_body(x_ref, w1_ref, g_ref):
    g_ref[...] = jnp.dot(x_ref[...], w1_ref[...],
                         preferred_element_type=jnp.float32)


def _tc_scale_body(g_ref, degT_ref, h_ref):
    h_ref[...] = g_ref[...] * _dinv_from(degT_ref)


def _tc_d_body(s_ref, h1_ref, degT_ref, b1_ref, w2_ref, out_ref):
    dinv = _dinv_from(degT_ref)
    s = s_ref[0, :_N] + s_ref[1, :_N] + h1_ref[...]
    z = jnp.maximum(dinv * s + b1_ref[...], 0.0)
    h2 = jnp.dot(z, w2_ref[...], preferred_element_type=jnp.float32)
    out_ref[:_N] = h2 * dinv
    out_ref[_N:] = jnp.zeros((_NPAD - _N, _CLS), jnp.float32)


def _tc_f_body(s2_ref, h2_ref, degT_ref, b2_ref, out_ref):
    dinv = _dinv_from(degT_ref)
    o = dinv * (s2_ref[0, :_N] + s2_ref[1, :_N] + h2_ref[:_N]) + b2_ref[...]
    m = jnp.max(o, axis=1, keepdims=True)
    lse = jnp.log(jnp.sum(jnp.exp(o - m), axis=1, keepdims=True)) + m
    out_ref[...] = o - lse


_tc_g = pl.pallas_call(
    _tc_g_body, out_shape=jax.ShapeDtypeStruct((_N, _H), jnp.float32))
_tc_scale = pl.pallas_call(
    _tc_scale_body, out_shape=jax.ShapeDtypeStruct((_N, _H), jnp.float32))
_tc_d = pl.pallas_call(
    _tc_d_body, out_shape=jax.ShapeDtypeStruct((_NPAD, _CLS), jnp.float32))
_tc_f = pl.pallas_call(
    _tc_f_body, out_shape=jax.ShapeDtypeStruct((_N, _CLS), jnp.float32))


@jax.jit
def _run(x, edge_index, W1, b1, W2, b2):
    ei = edge_index.astype(jnp.int32)

    def pad_edges(chunk, nchunk):
        pad = _NW * nchunk * chunk - _E
        pad_ids = jnp.arange(pad, dtype=jnp.int32)
        s = jnp.concatenate([ei[0], pad_ids % _N])
        t = jnp.concatenate([ei[1], _N + pad_ids % (_NPAD - _N)])
        return (s.reshape(_NW, nchunk, chunk), t.reshape(_NW, nchunk, chunk))

    src, dst = pad_edges(_CHUNK1, _NCHUNK1)
    src2, dst2 = pad_edges(_CHUNK2, _NCHUNK2)
    z_deg = jnp.zeros((_NPAD,), jnp.float32)
    z_h = jnp.zeros((_NPAD, _H), jnp.float32)
    z_c = jnp.zeros((_NPAD, _CLS), jnp.float32)

    g = _tc_g(x, W1)
    deg_parts = _deg_call(dst2, z_deg)
    degT = deg_parts[:, :_N].T

    h1t = _tc_scale(g, degT)
    s1 = _agg_call_h(h1t, src, dst, z_h)
    h2t = _tc_d(s1, h1t, degT, b1.reshape(1, _H), W2)
    s2 = _agg_call_c(h2t, src2, dst2, z_c)
    return _tc_f(s2, h2t, degT, b2.reshape(1, _CLS))


def kernel(x, edge_index, W1, b1, W2, b2):
    return _run(x, edge_index, W1, b1, W2, b2)
